# initial kernel scaffold (unmeasured)
import jax
import jax.numpy as jnp
from jax import lax
from jax.experimental import pallas as pl
from jax.experimental.pallas import tpu as pltpu

NZ = 4


def kernel(x, dy):
    k_per, m = x.shape
    _, f = dy.shape
    chunk = m // NZ

    def body(x_ref, dy_ref, out_ref, part_ref, comm_ref, send_sems, recv_sems):
        my_x = lax.axis_index("x")
        my_y = lax.axis_index("y")
        my_z = lax.axis_index("z")
        right = (my_x, my_y, (my_z + 1) % NZ)
        left = (my_x, my_y, (my_z - 1) % NZ)

        barrier_sem = pltpu.get_barrier_semaphore()
        for nbr in (left, right):
            pl.semaphore_signal(
                barrier_sem, inc=1,
                device_id=nbr, device_id_type=pl.DeviceIdType.MESH,
            )

        part_ref[:] = lax.dot_general(
            x_ref[:], dy_ref[:],
            dimension_numbers=(((0,), (0,)), ((), ())),
            preferred_element_type=jnp.float32,
        )

        pl.semaphore_wait(barrier_sem, 2)

        def rows(j):
            return pl.ds(j * chunk, chunk)

        rdma0 = pltpu.make_async_remote_copy(
            src_ref=part_ref.at[rows((my_z - 1) % NZ)],
            dst_ref=comm_ref.at[0],
            send_sem=send_sems.at[0],
            recv_sem=recv_sems.at[0],
            device_id=right,
            device_id_type=pl.DeviceIdType.MESH,
        )
        rdma0.start()
        rdma0.wait()

        for h in range(1, NZ - 1):
            c = (my_z - h - 1) % NZ
            comm_ref[h - 1] = comm_ref[h - 1] + part_ref[rows(c), :]
            rdma = pltpu.make_async_remote_copy(
                src_ref=comm_ref.at[h - 1],
                dst_ref=comm_ref.at[h],
                send_sem=send_sems.at[h],
                recv_sem=recv_sems.at[h],
                device_id=right,
                device_id_type=pl.DeviceIdType.MESH,
            )
            rdma.start()
            rdma.wait()

        out_ref[:] = comm_ref[NZ - 2] + part_ref[rows(my_z), :]

    return pl.pallas_call(
        body,
        out_shape=jax.ShapeDtypeStruct((chunk, f), jnp.float32),
        in_specs=[
            pl.BlockSpec(memory_space=pltpu.VMEM),
            pl.BlockSpec(memory_space=pltpu.VMEM),
        ],
        out_specs=pl.BlockSpec(memory_space=pltpu.VMEM),
        scratch_shapes=[
            pltpu.VMEM((m, f), jnp.float32),
            pltpu.VMEM((NZ - 1, chunk, f), jnp.float32),
            pltpu.SemaphoreType.DMA((NZ - 1,)),
            pltpu.SemaphoreType.DMA((NZ - 1,)),
        ],
        compiler_params=pltpu.CompilerParams(collective_id=0),
    )(x, dy)


# baseline (device time: 166871 ns/iter reference)
import jax
import jax.numpy as jnp
from jax import lax
from jax.experimental import pallas as pl
from jax.experimental.pallas import tpu as pltpu

NZ = 4


def kernel(x, dy):
    k_per, m = x.shape
    _, f = dy.shape
    chunk = m // NZ

    def body(x_ref, dy_ref, out_ref, part_ref, comm_ref, send_sems, recv_sems):
        my_x = lax.axis_index("x")
        my_y = lax.axis_index("y")
        my_z = lax.axis_index("z")
        right = (my_x, my_y, (my_z + 1) % NZ)
        left = (my_x, my_y, (my_z - 1) % NZ)

        barrier_sem = pltpu.get_barrier_semaphore()
        for nbr in (left, right):
            pl.semaphore_signal(
                barrier_sem, inc=1,
                device_id=nbr, device_id_type=pl.DeviceIdType.MESH,
            )

        part_ref[:] = lax.dot_general(
            x_ref[:], dy_ref[:],
            dimension_numbers=(((0,), (0,)), ((), ())),
            preferred_element_type=jnp.float32,
        )

        pl.semaphore_wait(barrier_sem, 2)

        def rows(j):
            return pl.ds(j * chunk, chunk)

        rdma0 = pltpu.make_async_remote_copy(
            src_ref=part_ref.at[rows((my_z - 1) % NZ)],
            dst_ref=comm_ref.at[0],
            send_sem=send_sems.at[0],
            recv_sem=recv_sems.at[0],
            device_id=right,
            device_id_type=pl.DeviceIdType.MESH,
        )
        rdma0.start()
        rdma0.wait()

        for h in range(1, NZ - 1):
            c = (my_z - h - 1) % NZ
            comm_ref[h - 1] = comm_ref[h - 1] + part_ref[rows(c), :]
            rdma = pltpu.make_async_remote_copy(
                src_ref=comm_ref.at[h - 1],
                dst_ref=comm_ref.at[h],
                send_sem=send_sems.at[h],
                recv_sem=recv_sems.at[h],
                device_id=right,
                device_id_type=pl.DeviceIdType.MESH,
            )
            rdma.start()
            rdma.wait()

        out_ref[:] = comm_ref[NZ - 2] + part_ref[rows(my_z), :]

    return pl.pallas_call(
        body,
        out_shape=jax.ShapeDtypeStruct((chunk, f), jnp.float32),
        in_specs=[
            pl.BlockSpec(memory_space=pltpu.VMEM),
            pl.BlockSpec(memory_space=pltpu.VMEM),
        ],
        out_specs=pl.BlockSpec(memory_space=pltpu.VMEM),
        scratch_shapes=[
            pltpu.VMEM((m, f), jnp.float32),
            pltpu.VMEM((NZ - 1, chunk, f), jnp.float32),
            pltpu.SemaphoreType.DMA((NZ - 1,)),
            pltpu.SemaphoreType.DMA((NZ - 1,)),
        ],
        compiler_params=pltpu.CompilerParams(
            collective_id=0,
            vmem_limit_bytes=100 * 1024 * 1024,
        ),
    )(x, dy)


# device time: 93436 ns/iter; 1.7859x vs baseline; 1.7859x over previous
import jax
import jax.numpy as jnp
from jax import lax
from jax.experimental import pallas as pl
from jax.experimental.pallas import tpu as pltpu

NZ = 4


def kernel(x, dy):
    k_per, m = x.shape
    _, f = dy.shape
    fq = f // 4
    chunk = m // NZ

    def body(x_ref, dy_hbm, out_ref, dyq_ref, part_ref, comm_ref,
             copy_sem, rs_send, rs_recv, ag_send, ag_recv):
        my_x = lax.axis_index("x")
        my_y = lax.axis_index("y")
        my_z = lax.axis_index("z")
        q = my_x + 2 * my_y
        right = (my_x, my_y, (my_z + 1) % NZ)
        left = (my_x, my_y, (my_z - 1) % NZ)
        xn = (1 - my_x, my_y, my_z)
        yn = (my_x, 1 - my_y, my_z)

        barrier_sem = pltpu.get_barrier_semaphore()
        for nbr in (left, right, xn, yn):
            pl.semaphore_signal(
                barrier_sem, inc=1,
                device_id=nbr, device_id_type=pl.DeviceIdType.MESH,
            )

        cp = pltpu.make_async_copy(
            dy_hbm.at[:, pl.ds(q * fq, fq)], dyq_ref, copy_sem)
        cp.start()
        cp.wait()

        part_ref[:] = lax.dot_general(
            x_ref[:], dyq_ref[:],
            dimension_numbers=(((0,), (0,)), ((), ())),
            preferred_element_type=jnp.float32,
        )

        pl.semaphore_wait(barrier_sem, 4)

        def rows(j):
            return pl.ds(j * chunk, chunk)

        rdma0 = pltpu.make_async_remote_copy(
            src_ref=part_ref.at[rows((my_z - 1) % NZ)],
            dst_ref=comm_ref.at[0],
            send_sem=rs_send.at[0],
            recv_sem=rs_recv.at[0],
            device_id=right,
            device_id_type=pl.DeviceIdType.MESH,
        )
        rdma0.start()
        rdma0.wait()

        for h in range(1, NZ - 1):
            c = (my_z - h - 1) % NZ
            comm_ref[h - 1] = comm_ref[h - 1] + part_ref[rows(c), :]
            rdma = pltpu.make_async_remote_copy(
                src_ref=comm_ref.at[h - 1],
                dst_ref=comm_ref.at[h],
                send_sem=rs_send.at[h],
                recv_sem=rs_recv.at[h],
                device_id=right,
                device_id_type=pl.DeviceIdType.MESH,
            )
            rdma.start()
            rdma.wait()

        qcols = pl.ds(q * fq, fq)
        out_ref[:, qcols] = comm_ref[NZ - 2] + part_ref[rows(my_z), :]

        rx = pltpu.make_async_remote_copy(
            src_ref=out_ref.at[:, qcols],
            dst_ref=out_ref.at[:, qcols],
            send_sem=ag_send.at[0],
            recv_sem=ag_recv.at[0],
            device_id=xn,
            device_id_type=pl.DeviceIdType.MESH,
        )
        rx.start()
        rx.wait()

        hcols = pl.ds(my_y * (2 * fq), 2 * fq)
        ry = pltpu.make_async_remote_copy(
            src_ref=out_ref.at[:, hcols],
            dst_ref=out_ref.at[:, hcols],
            send_sem=ag_send.at[1],
            recv_sem=ag_recv.at[1],
            device_id=yn,
            device_id_type=pl.DeviceIdType.MESH,
        )
        ry.start()
        ry.wait()

    return pl.pallas_call(
        body,
        out_shape=jax.ShapeDtypeStruct((chunk, f), jnp.float32),
        in_specs=[
            pl.BlockSpec(memory_space=pltpu.VMEM),
            pl.BlockSpec(memory_space=pl.ANY),
        ],
        out_specs=pl.BlockSpec(memory_space=pltpu.VMEM),
        scratch_shapes=[
            pltpu.VMEM((k_per, fq), jnp.float32),
            pltpu.VMEM((m, fq), jnp.float32),
            pltpu.VMEM((NZ - 1, chunk, fq), jnp.float32),
            pltpu.SemaphoreType.DMA,
            pltpu.SemaphoreType.DMA((NZ - 1,)),
            pltpu.SemaphoreType.DMA((NZ - 1,)),
            pltpu.SemaphoreType.DMA((2,)),
            pltpu.SemaphoreType.DMA((2,)),
        ],
        compiler_params=pltpu.CompilerParams(
            collective_id=0,
            vmem_limit_bytes=100 * 1024 * 1024,
        ),
    )(x, dy)


# device time: 67406 ns/iter; 2.4756x vs baseline; 1.3862x over previous
import jax
import jax.numpy as jnp
from jax import lax
from jax.experimental import pallas as pl
from jax.experimental.pallas import tpu as pltpu

NZ = 4
NSUB = 4


def kernel(x, dy):
    k_per, m = x.shape
    _, f = dy.shape
    fq = f // 4
    fs = fq // NSUB
    chunk = m // NZ

    def body(x_ref, dy_hbm, out_ref, dyq_ref, part_ref, comm_ref,
             copy_sem, rs_send, rs_recv,
             t1_send, t1_recv, t2_send, t2_recv, t3_send, t3_recv):
        my_x = lax.axis_index("x")
        my_y = lax.axis_index("y")
        my_z = lax.axis_index("z")
        q = my_x + 2 * my_y
        qx = (1 - my_x) + 2 * my_y
        right = (my_x, my_y, (my_z + 1) % NZ)
        left = (my_x, my_y, (my_z - 1) % NZ)
        xn = (1 - my_x, my_y, my_z)
        yn = (my_x, 1 - my_y, my_z)

        barrier_sem = pltpu.get_barrier_semaphore()
        for nbr in (left, right, xn, yn):
            pl.semaphore_signal(
                barrier_sem, inc=1,
                device_id=nbr, device_id_type=pl.DeviceIdType.MESH,
            )

        cp = pltpu.make_async_copy(
            dy_hbm.at[:, pl.ds(q * fq, fq)], dyq_ref, copy_sem)
        cp.start()

        def rows(j):
            return pl.ds(j * chunk, chunk)

        def scols(s):
            return pl.ds(s * fs, fs)

        def gemm_chunk(c):
            part_ref[rows(c), :] = lax.dot_general(
                x_ref[:, rows(c)], dyq_ref[:],
                dimension_numbers=(((0,), (0,)), ((), ())),
                preferred_element_type=jnp.float32,
            )

        cp.wait()
        gemm_chunk((my_z - 1) % NZ)

        pl.semaphore_wait(barrier_sem, 4)

        rs = {}
        for s in range(NSUB):
            r = pltpu.make_async_remote_copy(
                src_ref=part_ref.at[rows((my_z - 1) % NZ), scols(s)],
                dst_ref=comm_ref.at[s, 0],
                send_sem=rs_send.at[s, 0],
                recv_sem=rs_recv.at[s, 0],
                device_id=right,
                device_id_type=pl.DeviceIdType.MESH,
            )
            r.start()
            rs[(s, 0)] = r

        for j in range(2, NZ + 1):
            gemm_chunk((my_z - j) % NZ)

        for h in range(1, NZ - 1):
            c = (my_z - h - 1) % NZ
            for s in range(NSUB):
                rs[(s, h - 1)].wait()
                comm_ref[s, h - 1] = (
                    comm_ref[s, h - 1] + part_ref[rows(c), scols(s)])
                r = pltpu.make_async_remote_copy(
                    src_ref=comm_ref.at[s, h - 1],
                    dst_ref=comm_ref.at[s, h],
                    send_sem=rs_send.at[s, h],
                    recv_sem=rs_recv.at[s, h],
                    device_id=right,
                    device_id_type=pl.DeviceIdType.MESH,
                )
                r.start()
                rs[(s, h)] = r

        qc = q * fq
        qxc = qx * fq
        t1 = {}
        t2 = {}
        t3 = {}
        for s in range(NSUB):
            rs[(s, NZ - 2)].wait()
            mycols = pl.ds(qc + s * fs, fs)
            out_ref[:, mycols] = (
                comm_ref[s, NZ - 2] + part_ref[rows(my_z), scols(s)])
            a = pltpu.make_async_remote_copy(
                src_ref=out_ref.at[:, mycols],
                dst_ref=out_ref.at[:, mycols],
                send_sem=t1_send.at[s],
                recv_sem=t1_recv.at[s],
                device_id=xn,
                device_id_type=pl.DeviceIdType.MESH,
            )
            a.start()
            t1[s] = a
            b = pltpu.make_async_remote_copy(
                src_ref=out_ref.at[:, mycols],
                dst_ref=out_ref.at[:, mycols],
                send_sem=t2_send.at[s],
                recv_sem=t2_recv.at[s],
                device_id=yn,
                device_id_type=pl.DeviceIdType.MESH,
            )
            b.start()
            t2[s] = b

        for s in range(NSUB):
            t1[s].wait_recv()
            xcols = pl.ds(qxc + s * fs, fs)
            c = pltpu.make_async_remote_copy(
                src_ref=out_ref.at[:, xcols],
                dst_ref=out_ref.at[:, xcols],
                send_sem=t3_send.at[s],
                recv_sem=t3_recv.at[s],
                device_id=yn,
                device_id_type=pl.DeviceIdType.MESH,
            )
            c.start()
            t3[s] = c

        for s in range(NSUB):
            t1[s].wait_send()
            t2[s].wait()
            t3[s].wait()

    return pl.pallas_call(
        body,
        out_shape=jax.ShapeDtypeStruct((chunk, f), jnp.float32),
        in_specs=[
            pl.BlockSpec(memory_space=pltpu.VMEM),
            pl.BlockSpec(memory_space=pl.ANY),
        ],
        out_specs=pl.BlockSpec(memory_space=pltpu.VMEM),
        scratch_shapes=[
            pltpu.VMEM((k_per, fq), jnp.float32),
            pltpu.VMEM((m, fq), jnp.float32),
            pltpu.VMEM((NSUB, NZ - 1, chunk, fs), jnp.float32),
            pltpu.SemaphoreType.DMA,
            pltpu.SemaphoreType.DMA((NSUB, NZ - 1)),
            pltpu.SemaphoreType.DMA((NSUB, NZ - 1)),
            pltpu.SemaphoreType.DMA((NSUB,)),
            pltpu.SemaphoreType.DMA((NSUB,)),
            pltpu.SemaphoreType.DMA((NSUB,)),
            pltpu.SemaphoreType.DMA((NSUB,)),
            pltpu.SemaphoreType.DMA((NSUB,)),
            pltpu.SemaphoreType.DMA((NSUB,)),
        ],
        compiler_params=pltpu.CompilerParams(
            collective_id=0,
            vmem_limit_bytes=100 * 1024 * 1024,
        ),
    )(x, dy)


# device time: 66413 ns/iter; 2.5126x vs baseline; 1.0150x over previous
import jax
import jax.numpy as jnp
from jax import lax
from jax.experimental import pallas as pl
from jax.experimental.pallas import tpu as pltpu

NZ = 4
NSUB = 8


def kernel(x, dy):
    k_per, m = x.shape
    _, f = dy.shape
    fq = f // 4
    fs = fq // NSUB
    chunk = m // NZ

    def body(x_ref, dy_hbm, out_ref, dyq_ref, part_ref, comm_ref,
             copy_sem, rs_send, rs_recv,
             t1_send, t1_recv, t2_send, t2_recv, t3_send, t3_recv):
        my_x = lax.axis_index("x")
        my_y = lax.axis_index("y")
        my_z = lax.axis_index("z")
        q = my_x + 2 * my_y
        qx = (1 - my_x) + 2 * my_y
        right = (my_x, my_y, (my_z + 1) % NZ)
        left = (my_x, my_y, (my_z - 1) % NZ)
        xn = (1 - my_x, my_y, my_z)
        yn = (my_x, 1 - my_y, my_z)

        barrier_sem = pltpu.get_barrier_semaphore()
        for nbr in (left, right, xn, yn):
            pl.semaphore_signal(
                barrier_sem, inc=1,
                device_id=nbr, device_id_type=pl.DeviceIdType.MESH,
            )

        cp = pltpu.make_async_copy(
            dy_hbm.at[:, pl.ds(q * fq, fq)], dyq_ref, copy_sem)
        cp.start()

        def rows(j):
            return pl.ds(j * chunk, chunk)

        def scols(s):
            return pl.ds(s * fs, fs)

        def gemm_chunk(c):
            part_ref[rows(c), :] = lax.dot_general(
                x_ref[:, rows(c)], dyq_ref[:],
                dimension_numbers=(((0,), (0,)), ((), ())),
                preferred_element_type=jnp.float32,
            )

        cp.wait()
        gemm_chunk((my_z - 1) % NZ)

        pl.semaphore_wait(barrier_sem, 4)

        rs = {}
        for s in range(NSUB):
            r = pltpu.make_async_remote_copy(
                src_ref=part_ref.at[rows((my_z - 1) % NZ), scols(s)],
                dst_ref=comm_ref.at[s, 0],
                send_sem=rs_send.at[s, 0],
                recv_sem=rs_recv.at[s, 0],
                device_id=right,
                device_id_type=pl.DeviceIdType.MESH,
            )
            r.start()
            rs[(s, 0)] = r

        for j in range(2, NZ + 1):
            gemm_chunk((my_z - j) % NZ)

        for h in range(1, NZ - 1):
            c = (my_z - h - 1) % NZ
            for s in range(NSUB):
                rs[(s, h - 1)].wait()
                comm_ref[s, h - 1] = (
                    comm_ref[s, h - 1] + part_ref[rows(c), scols(s)])
                r = pltpu.make_async_remote_copy(
                    src_ref=comm_ref.at[s, h - 1],
                    dst_ref=comm_ref.at[s, h],
                    send_sem=rs_send.at[s, h],
                    recv_sem=rs_recv.at[s, h],
                    device_id=right,
                    device_id_type=pl.DeviceIdType.MESH,
                )
                r.start()
                rs[(s, h)] = r

        qc = q * fq
        qxc = qx * fq
        t1 = {}
        t2 = {}
        t3 = {}
        for s in range(NSUB):
            rs[(s, NZ - 2)].wait()
            mycols = pl.ds(qc + s * fs, fs)
            out_ref[:, mycols] = (
                comm_ref[s, NZ - 2] + part_ref[rows(my_z), scols(s)])
            a = pltpu.make_async_remote_copy(
                src_ref=out_ref.at[:, mycols],
                dst_ref=out_ref.at[:, mycols],
                send_sem=t1_send.at[s],
                recv_sem=t1_recv.at[s],
                device_id=xn,
                device_id_type=pl.DeviceIdType.MESH,
            )
            a.start()
            t1[s] = a
            b = pltpu.make_async_remote_copy(
                src_ref=out_ref.at[:, mycols],
                dst_ref=out_ref.at[:, mycols],
                send_sem=t2_send.at[s],
                recv_sem=t2_recv.at[s],
                device_id=yn,
                device_id_type=pl.DeviceIdType.MESH,
            )
            b.start()
            t2[s] = b

        for s in range(NSUB):
            t1[s].wait_recv()
            xcols = pl.ds(qxc + s * fs, fs)
            c = pltpu.make_async_remote_copy(
                src_ref=out_ref.at[:, xcols],
                dst_ref=out_ref.at[:, xcols],
                send_sem=t3_send.at[s],
                recv_sem=t3_recv.at[s],
                device_id=yn,
                device_id_type=pl.DeviceIdType.MESH,
            )
            c.start()
            t3[s] = c

        for s in range(NSUB):
            t1[s].wait_send()
            t2[s].wait()
            t3[s].wait()

    return pl.pallas_call(
        body,
        out_shape=jax.ShapeDtypeStruct((chunk, f), jnp.float32),
        in_specs=[
            pl.BlockSpec(memory_space=pltpu.VMEM),
            pl.BlockSpec(memory_space=pl.ANY),
        ],
        out_specs=pl.BlockSpec(memory_space=pltpu.VMEM),
        scratch_shapes=[
            pltpu.VMEM((k_per, fq), jnp.float32),
            pltpu.VMEM((m, fq), jnp.float32),
            pltpu.VMEM((NSUB, NZ - 1, chunk, fs), jnp.float32),
            pltpu.SemaphoreType.DMA,
            pltpu.SemaphoreType.DMA((NSUB, NZ - 1)),
            pltpu.SemaphoreType.DMA((NSUB, NZ - 1)),
            pltpu.SemaphoreType.DMA((NSUB,)),
            pltpu.SemaphoreType.DMA((NSUB,)),
            pltpu.SemaphoreType.DMA((NSUB,)),
            pltpu.SemaphoreType.DMA((NSUB,)),
            pltpu.SemaphoreType.DMA((NSUB,)),
            pltpu.SemaphoreType.DMA((NSUB,)),
        ],
        compiler_params=pltpu.CompilerParams(
            collective_id=0,
            vmem_limit_bytes=100 * 1024 * 1024,
        ),
    )(x, dy)


# device time: 65458 ns/iter; 2.5493x vs baseline; 1.0146x over previous
import jax
import jax.numpy as jnp
from jax import lax
from jax.experimental import pallas as pl
from jax.experimental.pallas import tpu as pltpu

NZ = 4
NSUB = 8


def kernel(x, dy):
    k_per, m = x.shape
    _, f = dy.shape
    fq = f // 4
    fs = fq // NSUB
    chunk = m // NZ

    def body(x_hbm, dy_hbm, out_ref, x_ref, dyq_ref, part_ref, comm_ref,
             xcopy_sem, copy_sem, rs_send, rs_recv,
             t1_send, t1_recv, t2_send, t2_recv, t3_send, t3_recv):
        my_x = lax.axis_index("x")
        my_y = lax.axis_index("y")
        my_z = lax.axis_index("z")
        q = my_x + 2 * my_y
        qx = (1 - my_x) + 2 * my_y
        right = (my_x, my_y, (my_z + 1) % NZ)
        left = (my_x, my_y, (my_z - 1) % NZ)
        xn = (1 - my_x, my_y, my_z)
        yn = (my_x, 1 - my_y, my_z)

        barrier_sem = pltpu.get_barrier_semaphore()
        for nbr in (left, right, xn, yn):
            pl.semaphore_signal(
                barrier_sem, inc=1,
                device_id=nbr, device_id_type=pl.DeviceIdType.MESH,
            )

        xcp = pltpu.make_async_copy(x_hbm, x_ref, xcopy_sem)
        xcp.start()
        cp = pltpu.make_async_copy(
            dy_hbm.at[:, pl.ds(q * fq, fq)], dyq_ref, copy_sem)
        cp.start()

        def rows(j):
            return pl.ds(j * chunk, chunk)

        def scols(s):
            return pl.ds(s * fs, fs)

        def gemm_chunk(c):
            part_ref[rows(c), :] = lax.dot_general(
                x_ref[:, rows(c)], dyq_ref[:],
                dimension_numbers=(((0,), (0,)), ((), ())),
                preferred_element_type=jnp.float32,
            )

        xcp.wait()
        cp.wait()
        gemm_chunk((my_z - 1) % NZ)

        pl.semaphore_wait(barrier_sem, 4)

        rs = {}
        for s in range(NSUB):
            r = pltpu.make_async_remote_copy(
                src_ref=part_ref.at[rows((my_z - 1) % NZ), scols(s)],
                dst_ref=comm_ref.at[s, 0],
                send_sem=rs_send.at[s, 0],
                recv_sem=rs_recv.at[s, 0],
                device_id=right,
                device_id_type=pl.DeviceIdType.MESH,
            )
            r.start()
            rs[(s, 0)] = r

        for j in range(2, NZ + 1):
            gemm_chunk((my_z - j) % NZ)

        for h in range(1, NZ - 1):
            c = (my_z - h - 1) % NZ
            for s in range(NSUB):
                rs[(s, h - 1)].wait()
                comm_ref[s, h - 1] = (
                    comm_ref[s, h - 1] + part_ref[rows(c), scols(s)])
                r = pltpu.make_async_remote_copy(
                    src_ref=comm_ref.at[s, h - 1],
                    dst_ref=comm_ref.at[s, h],
                    send_sem=rs_send.at[s, h],
                    recv_sem=rs_recv.at[s, h],
                    device_id=right,
                    device_id_type=pl.DeviceIdType.MESH,
                )
                r.start()
                rs[(s, h)] = r

        qc = q * fq
        qxc = qx * fq
        t1 = {}
        t2 = {}
        t3 = {}
        for s in range(NSUB):
            rs[(s, NZ - 2)].wait()
            mycols = pl.ds(qc + s * fs, fs)
            out_ref[:, mycols] = (
                comm_ref[s, NZ - 2] + part_ref[rows(my_z), scols(s)])
            a = pltpu.make_async_remote_copy(
                src_ref=out_ref.at[:, mycols],
                dst_ref=out_ref.at[:, mycols],
                send_sem=t1_send.at[s],
                recv_sem=t1_recv.at[s],
                device_id=xn,
                device_id_type=pl.DeviceIdType.MESH,
            )
            a.start()
            t1[s] = a
            b = pltpu.make_async_remote_copy(
                src_ref=out_ref.at[:, mycols],
                dst_ref=out_ref.at[:, mycols],
                send_sem=t2_send.at[s],
                recv_sem=t2_recv.at[s],
                device_id=yn,
                device_id_type=pl.DeviceIdType.MESH,
            )
            b.start()
            t2[s] = b

        for s in range(NSUB):
            t1[s].wait_recv()
            xcols = pl.ds(qxc + s * fs, fs)
            c = pltpu.make_async_remote_copy(
                src_ref=out_ref.at[:, xcols],
                dst_ref=out_ref.at[:, xcols],
                send_sem=t3_send.at[s],
                recv_sem=t3_recv.at[s],
                device_id=yn,
                device_id_type=pl.DeviceIdType.MESH,
            )
            c.start()
            t3[s] = c

        for s in range(NSUB):
            t1[s].wait_send()
            t2[s].wait()
            t3[s].wait()

    return pl.pallas_call(
        body,
        out_shape=jax.ShapeDtypeStruct((chunk, f), jnp.float32),
        in_specs=[
            pl.BlockSpec(memory_space=pl.ANY),
            pl.BlockSpec(memory_space=pl.ANY),
        ],
        out_specs=pl.BlockSpec(memory_space=pltpu.VMEM),
        scratch_shapes=[
            pltpu.VMEM((k_per, m), jnp.float32),
            pltpu.VMEM((k_per, fq), jnp.float32),
            pltpu.VMEM((m, fq), jnp.float32),
            pltpu.VMEM((NSUB, NZ - 1, chunk, fs), jnp.float32),
            pltpu.SemaphoreType.DMA,
            pltpu.SemaphoreType.DMA,
            pltpu.SemaphoreType.DMA((NSUB, NZ - 1)),
            pltpu.SemaphoreType.DMA((NSUB, NZ - 1)),
            pltpu.SemaphoreType.DMA((NSUB,)),
            pltpu.SemaphoreType.DMA((NSUB,)),
            pltpu.SemaphoreType.DMA((NSUB,)),
            pltpu.SemaphoreType.DMA((NSUB,)),
            pltpu.SemaphoreType.DMA((NSUB,)),
            pltpu.SemaphoreType.DMA((NSUB,)),
        ],
        compiler_params=pltpu.CompilerParams(
            collective_id=0,
            vmem_limit_bytes=100 * 1024 * 1024,
        ),
    )(x, dy)
